# baseline (speedup 1.0000x reference)
import functools

import jax
import jax.numpy as jnp
from jax import lax
from jax.experimental import pallas as pl
from jax.experimental.pallas import tpu as pltpu
from jax.experimental.pallas import tpu_sc as plsc

_F32 = jnp.float32
_PREC = lax.Precision.DEFAULT


def _dot(a, b):
    return lax.dot(a, b, precision=_PREC, preferred_element_type=_F32)


def _vmlp_body(x_ref, v_ref, wi, bi, w1, b1, w2, b2, wo, bo, o_ref):
    x = x_ref[...]
    h = jnp.maximum(_dot(x, wi[...]) + bi[...], 0.0)
    h = h + jnp.maximum(_dot(h, w1[...]) + b1[...], 0.0)
    h = h + jnp.maximum(_dot(h, w2[...]) + b2[...], 0.0)
    o_ref[:, 0:256] = _dot(h, wo[...]) + bo[...]
    o_ref[:, 256:384] = jnp.pad(v_ref[...], ((0, 0), (0, 124)))


def _vertex_mlp(xp, vp, params):
    npad = xp.shape[0]
    blk = 1024
    grid = (npad // blk,)

    def wspec(shape):
        return pl.BlockSpec(shape, lambda i: (0, 0))

    args = [
        params['Wd_in'], params['bd_in'].reshape(1, -1),
        params['Wd1'], params['bd1'].reshape(1, -1),
        params['Wd2'], params['bd2'].reshape(1, -1),
        params['Wd_out'], params['bd_out'].reshape(1, -1),
    ]
    return pl.pallas_call(
        _vmlp_body,
        grid=grid,
        in_specs=[pl.BlockSpec((blk, 128), lambda i: (i, 0)),
                  pl.BlockSpec((blk, 4), lambda i: (i, 0))] +
                 [wspec(a.shape) for a in args],
        out_specs=pl.BlockSpec((blk, 384), lambda i: (i, 0)),
        out_shape=jax.ShapeDtypeStruct((npad, 384), _F32),
    )(xp, vp, *args)


def _rmlp_body(s_ref, w0, b0, w1, b1, w2, b2, w3, b3, w4, b4, w5, b5, o_ref):
    h = jnp.maximum(_dot(s_ref[...], w0[...]) + b0[...], 0.0)
    h = jnp.maximum(_dot(h, w1[...]) + b1[...], 0.0)
    h = jnp.maximum(_dot(h, w2[...]) + b2[...], 0.0)
    h = jnp.maximum(_dot(h, w3[...]) + b3[...], 0.0)
    h = jnp.maximum(_dot(h, w4[...]) + b4[...], 0.0)
    ft = lax.dot_general(w5[...], h, (((0,), (1,)), ((), ())),
                         precision=_PREC, preferred_element_type=_F32)
    o_ref[...] = ft + b5[...]


def _refine_mlp(s, params):
    fp = s.shape[0]
    blk = 1024
    grid = (fp // blk,)
    w5 = jnp.pad(params['Wr5'], ((0, 0), (0, 4)))
    b5 = jnp.pad(params['br5'], ((0, 4),))
    args = [
        params['Wr0'] * (1.0 / 3.0),
        params['br0'].reshape(1, -1),
        params['Wr1'], params['br1'].reshape(1, -1),
        params['Wr2'], params['br2'].reshape(1, -1),
        params['Wr3'], params['br3'].reshape(1, -1),
        params['Wr4'], params['br4'].reshape(1, -1),
        w5, b5.reshape(-1, 1),
    ]

    def wspec(shape):
        return pl.BlockSpec(shape, lambda i: (0, 0))

    return pl.pallas_call(
        _rmlp_body,
        grid=grid,
        in_specs=[pl.BlockSpec((blk, 256), lambda i: (i, 0))] +
                 [wspec(a.shape) for a in args],
        out_specs=pl.BlockSpec((16, blk), lambda i: (0, i)),
        out_shape=jax.ShapeDtypeStruct((16, fp), _F32),
    )(s, *args)


def _sym4_sq_norm(b):
    b00, b01, b02, b03, b11, b12, b13, b22, b23, b33 = b
    c00 = b00 * b00 + b01 * b01 + b02 * b02 + b03 * b03
    c01 = b00 * b01 + b01 * b11 + b02 * b12 + b03 * b13
    c02 = b00 * b02 + b01 * b12 + b02 * b22 + b03 * b23
    c03 = b00 * b03 + b01 * b13 + b02 * b23 + b03 * b33
    c11 = b01 * b01 + b11 * b11 + b12 * b12 + b13 * b13
    c12 = b01 * b02 + b11 * b12 + b12 * b22 + b13 * b23
    c13 = b01 * b03 + b11 * b13 + b12 * b23 + b13 * b33
    c22 = b02 * b02 + b12 * b12 + b22 * b22 + b23 * b23
    c23 = b02 * b03 + b12 * b13 + b22 * b23 + b23 * b33
    c33 = b03 * b03 + b13 * b13 + b23 * b23 + b33 * b33
    tr = c00 + c11 + c22 + c33
    r = 1.0 / jnp.maximum(tr, 1e-30)
    return tuple(v * r for v in (c00, c01, c02, c03, c11, c12, c13, c22, c23, c33))


def _proc_body(f_real, f_ref, p0_ref, p1_ref, p2_ref, r_ref, tp_ref, sc_ref):
    m = [f_ref[k] for k in range(9)]
    t = [f_ref[9 + k] for k in range(3)]
    m00, m01, m02, m10, m11, m12, m20, m21, m22 = m
    fro2 = m00 * m00
    for mi in m[1:]:
        fro2 = fro2 + mi * mi
    sig = jnp.sqrt(3.0 * fro2)
    k00 = m00 + m11 + m22
    k01 = m21 - m12
    k02 = m02 - m20
    k03 = m10 - m01
    k11 = m00 - m11 - m22
    k12 = m01 + m10
    k13 = m02 + m20
    k22 = m11 - m00 - m22
    k23 = m12 + m21
    k33 = m22 - m00 - m11
    r0 = 1.0 / jnp.maximum(4.0 * sig, 1e-30)
    b = tuple(v * r0 for v in (k00 + sig, k01, k02, k03, k11 + sig, k12, k13,
                               k22 + sig, k23, k33 + sig))
    for _ in range(12):
        b = _sym4_sq_norm(b)
    b00, b01, b02, b03, b11, b12, b13, b22, b23, b33 = b
    c01 = b00 >= b11
    a0 = jnp.where(c01, b00, b01)
    a1 = jnp.where(c01, b01, b11)
    a2 = jnp.where(c01, b02, b12)
    a3 = jnp.where(c01, b03, b13)
    da = jnp.where(c01, b00, b11)
    c23 = b22 >= b33
    e0 = jnp.where(c23, b02, b03)
    e1 = jnp.where(c23, b12, b13)
    e2 = jnp.where(c23, b22, b23)
    e3 = jnp.where(c23, b23, b33)
    de = jnp.where(c23, b22, b33)
    cab = da >= de
    qw = jnp.where(cab, a0, e0)
    qx = jnp.where(cab, a1, e1)
    qy = jnp.where(cab, a2, e2)
    qz = jnp.where(cab, a3, e3)
    inn = qw * qw + qx * qx + qy * qy + qz * qz
    s = lax.rsqrt(jnp.maximum(inn, 1e-30))
    w, x, y, z = qw * s, qx * s, qy * s, qz * s
    r00 = 1.0 - 2.0 * (y * y + z * z)
    r01 = 2.0 * (x * y - w * z)
    r02 = 2.0 * (x * z + w * y)
    r10 = 2.0 * (x * y + w * z)
    r11 = 1.0 - 2.0 * (x * x + z * z)
    r12 = 2.0 * (y * z - w * x)
    r20 = 2.0 * (x * z - w * y)
    r21 = 2.0 * (y * z + w * x)
    r22 = 1.0 - 2.0 * (x * x + y * y)
    rm = [r00, r01, r02, r10, r11, r12, r20, r21, r22]
    r_ref[...] = jnp.stack(rm, axis=-1)
    gi = pl.program_id(0)
    row = lax.broadcasted_iota(jnp.int32, (8, 128), 0)
    col = lax.broadcasted_iota(jnp.int32, (8, 128), 1)
    valid = ((gi * 1024 + row * 128 + col) < f_real).astype(_F32)
    ps = [p0_ref, p1_ref, p2_ref]
    tp = [[None] * 3 for _ in range(3)]
    for i in range(3):
        for k in range(3):
            tp[i][k] = (ps[i][:, :, 0] * rm[k] + ps[i][:, :, 1] * rm[3 + k] +
                        ps[i][:, :, 2] * rm[6 + k] + t[k])
    tp_ref[...] = jnp.stack(
        [tp[i][k] for i in range(3) for k in range(3)], axis=-1)
    for k in range(3):
        for i in range(3):
            sc_ref[k * 3 + i] = tp[i][k] * valid
    for i in range(3):
        sc_ref[9 + i] = valid


def _procrustes(ft, p0, p1, p2, f_real):
    nblk = ft.shape[1]
    nb = nblk // 8
    grid = (nb,)
    pspec = pl.BlockSpec((8, 128, 16), lambda i: (i, 0, 0))
    return pl.pallas_call(
        functools.partial(_proc_body, f_real),
        grid=grid,
        in_specs=[pl.BlockSpec((16, 8, 128), lambda i: (0, i, 0)),
                  pspec, pspec, pspec],
        out_specs=[pl.BlockSpec((8, 128, 9), lambda i: (i, 0, 0)),
                   pl.BlockSpec((8, 128, 9), lambda i: (i, 0, 0)),
                   pl.BlockSpec((12, 8, 128), lambda i: (0, i, 0))],
        out_shape=[jax.ShapeDtypeStruct((nblk, 128, 9), _F32),
                   jax.ShapeDtypeStruct((nblk, 128, 9), _F32),
                   jax.ShapeDtypeStruct((12, nblk, 128), _F32)],
    )(ft, p0, p1, p2)


def _comb_body(p_ref, o_ref):
    cnt = jnp.maximum(p_ref[0, 3, :] + p_ref[1, 3, :], 1.0)
    for k in range(3):
        o_ref[k] = (p_ref[0, k, :] + p_ref[1, k, :]) / cnt


def _combine(parts):
    nv = parts.shape[2]
    blk = 2048
    grid = (nv // blk,)
    return pl.pallas_call(
        _comb_body,
        grid=grid,
        in_specs=[pl.BlockSpec((2, 4, blk), lambda i: (0, 0, i))],
        out_specs=pl.BlockSpec((3, blk), lambda i: (0, i)),
        out_shape=jax.ShapeDtypeStruct((3, nv), _F32),
    )(parts)


def _face_gather(x, facep):
    fp = facep.shape[1]
    face_flat = facep.reshape(3 * fp)
    ch = 64
    nw = 32
    fw = fp // nw
    nch = fw // ch
    mesh = plsc.VectorSubcoreMesh(core_axis_name="c", subcore_axis_name="s")

    @functools.partial(
        pl.kernel, mesh=mesh,
        out_type=[jax.ShapeDtypeStruct((fp, 256), _F32),
                  jax.ShapeDtypeStruct((3, fp, 16), _F32)],
        scratch_types=[pltpu.VMEM((ch,), jnp.int32),
                       pltpu.VMEM((ch,), jnp.int32),
                       pltpu.VMEM((ch,), jnp.int32),
                       pltpu.VMEM((ch, 384), _F32),
                       pltpu.VMEM((ch, 384), _F32),
                       pltpu.VMEM((ch, 384), _F32),
                       pltpu.VMEM((ch, 256), _F32),
                       pltpu.VMEM((ch, 16), _F32),
                       pltpu.VMEM((ch, 16), _F32),
                       pltpu.VMEM((ch, 16), _F32),
                       pltpu.SemaphoreType.DMA],
    )
    def k(x_hbm, face_hbm, s_out, p_out, i0, i1, i2, b0, b1, b2, sbuf,
          v0, v1, v2, sem):
        c = lax.axis_index("c")
        s = lax.axis_index("s")
        wid = s * 2 + c
        idxs = [i0, i1, i2]
        bufs = [b0, b1, b2]
        vrs = [v0, v1, v2]

        def chunk(i, carry):
            base = wid * fw + i * ch
            for j in range(3):
                pltpu.sync_copy(face_hbm.at[pl.ds(j * fp + base, ch)], idxs[j])
            handles = [pltpu.async_copy(x_hbm.at[idxs[j]], bufs[j], sem)
                       for j in range(3)]
            for h in handles:
                h.wait()

            def rows(r2, carry2):
                for rr in range(2):
                    r = r2 * 2 + rr
                    for g in range(256 // 16):
                        sl = pl.ds(g * 16, 16)
                        sbuf[r, sl] = b0[r, sl] + b1[r, sl] + b2[r, sl]
                    for j in range(3):
                        vrs[j][r, :] = bufs[j][r, pl.ds(256, 16)]
                return carry2

            lax.fori_loop(0, ch // 2, rows, 0)
            pltpu.sync_copy(sbuf, s_out.at[pl.ds(base, ch)])
            for j in range(3):
                pltpu.sync_copy(vrs[j], p_out.at[j, pl.ds(base, ch)])
            return carry

        lax.fori_loop(0, nch, chunk, 0)

    return k(x, face_flat)


def _vertex_scatter(sdata_flat, sidx, nv):
    rp = sdata_flat.shape[0] // 4
    ch = 2048
    half = rp // 2
    nchunk = half // ch
    vt = nv // 16
    mesh = plsc.VectorSubcoreMesh(core_axis_name="c", subcore_axis_name="s")

    @functools.partial(
        pl.kernel, mesh=mesh,
        compiler_params=pltpu.CompilerParams(needs_layout_passes=False),
        out_type=jax.ShapeDtypeStruct((8 * nv,), _F32),
        scratch_types=[pltpu.VMEM((ch,), jnp.int32),
                       pltpu.VMEM((4 * ch,), _F32),
                       pltpu.VMEM((4 * vt,), _F32)],
    )
    def k(data_hbm, idx_hbm, out_hbm, idx_v, dbuf, acc):
        c = lax.axis_index("c")
        s = lax.axis_index("s")
        vbase = s * vt

        def z(i, carry):
            acc[pl.ds(i * 16, 16)] = jnp.zeros((16,), _F32)
            return carry

        lax.fori_loop(0, (4 * vt) // 16, z, 0)

        def chunk(i, carry):
            base = c * half + i * ch
            pltpu.sync_copy(idx_hbm.at[pl.ds(base, ch)], idx_v)
            for kk in range(4):
                pltpu.sync_copy(data_hbm.at[pl.ds(kk * rp + base, ch)],
                                dbuf.at[pl.ds(kk * ch, ch)])

            def g(j, carry2):
                iv = idx_v[pl.ds(j * 16, 16)]
                local = iv - vbase
                msk = (local >= 0) & (local < vt)
                loc = jnp.minimum(jnp.maximum(local, 0), vt - 1)
                for kk in range(4):
                    dk = dbuf[pl.ds(kk * ch + j * 16, 16)]
                    plsc.addupdate_scatter(acc, [loc + kk * vt], dk, mask=msk)
                return carry2

            lax.fori_loop(0, ch // 16, g, 0)
            return carry

        lax.fori_loop(0, nchunk, chunk, 0)
        for kk in range(4):
            pltpu.sync_copy(
                acc.at[pl.ds(kk * vt, vt)],
                out_hbm.at[pl.ds((c * 4 + kk) * nv + s * vt, vt)])

    return k(sdata_flat, sidx)


def kernel(pos, x_in, face, params):
    n = pos.shape[1]
    f = face.shape[1]
    blk = 1024
    npad = ((n + blk - 1) // blk) * blk
    fp = ((f + 4095) // 4096) * 4096

    xp = jnp.pad(x_in.reshape(n, 128), ((0, npad - n), (0, 0)))
    verts = pos.reshape(n, 3)
    vp = jnp.pad(verts, ((0, npad - n), (0, 1)))
    x = _vertex_mlp(xp, vp, params)

    facep = jnp.pad(face, ((0, 0), (0, fp - f)))

    ssum, prism = _face_gather(x, facep)

    fvec_cm = _refine_mlp(ssum, params)

    ft = fvec_cm.reshape(16, fp // 128, 128)
    pr = prism.reshape(3, fp // 128, 128, 16)
    r_rm, tp_rm, out12 = _procrustes(ft, pr[0], pr[1], pr[2], f)

    rmat = r_rm.reshape(fp, 9)[:f].reshape(f, 3, 3)
    tprism = tp_rm.reshape(fp, 9)[:f].reshape(f, 3, 3)

    sdata_flat = out12.reshape(12 * fp)
    sidx = facep.reshape(3 * fp)
    nvp = ((n + 16383) // 16384) * 16384

    parts = _vertex_scatter(sdata_flat, sidx, nvp)
    vcomb = _combine(parts.reshape(2, 4, nvp))

    return vcomb.T[:n].reshape(1, n, 3), tprism, rmat


# speedup vs baseline: 1.0727x; 1.0727x over previous
import functools

import jax
import jax.numpy as jnp
from jax import lax
from jax.experimental import pallas as pl
from jax.experimental.pallas import tpu as pltpu
from jax.experimental.pallas import tpu_sc as plsc

_F32 = jnp.float32
_PREC = lax.Precision.DEFAULT


def _dot(a, b):
    return lax.dot(a, b, precision=_PREC, preferred_element_type=_F32)


def _vmlp_body(x_ref, v_ref, wi, bi, w1, b1, w2, b2, wo, bo, o_ref):
    x = x_ref[...]
    h = jnp.maximum(_dot(x, wi[...]) + bi[...], 0.0)
    h = h + jnp.maximum(_dot(h, w1[...]) + b1[...], 0.0)
    h = h + jnp.maximum(_dot(h, w2[...]) + b2[...], 0.0)
    o_ref[:, 0:256] = _dot(h, wo[...]) + bo[...]
    o_ref[:, 256:384] = jnp.pad(v_ref[...], ((0, 0), (0, 124)))


def _vertex_mlp(xp, vp, params):
    npad = xp.shape[0]
    blk = 1024
    grid = (npad // blk,)

    def wspec(shape):
        return pl.BlockSpec(shape, lambda i: (0, 0))

    args = [
        params['Wd_in'], params['bd_in'].reshape(1, -1),
        params['Wd1'], params['bd1'].reshape(1, -1),
        params['Wd2'], params['bd2'].reshape(1, -1),
        params['Wd_out'], params['bd_out'].reshape(1, -1),
    ]
    return pl.pallas_call(
        _vmlp_body,
        grid=grid,
        in_specs=[pl.BlockSpec((blk, 128), lambda i: (i, 0)),
                  pl.BlockSpec((blk, 4), lambda i: (i, 0))] +
                 [wspec(a.shape) for a in args],
        out_specs=pl.BlockSpec((blk, 384), lambda i: (i, 0)),
        out_shape=jax.ShapeDtypeStruct((npad, 384), _F32),
    )(xp, vp, *args)


def _rmlp_body(s_ref, w0, b0, w1, b1, w2, b2, w3, b3, w4, b4, w5, b5, o_ref):
    h = jnp.maximum(_dot(s_ref[...], w0[...]) + b0[...], 0.0)
    h = jnp.maximum(_dot(h, w1[...]) + b1[...], 0.0)
    h = jnp.maximum(_dot(h, w2[...]) + b2[...], 0.0)
    h = jnp.maximum(_dot(h, w3[...]) + b3[...], 0.0)
    h = jnp.maximum(_dot(h, w4[...]) + b4[...], 0.0)
    ft = lax.dot_general(w5[...], h, (((0,), (1,)), ((), ())),
                         precision=_PREC, preferred_element_type=_F32)
    o_ref[...] = ft + b5[...]


def _refine_mlp(s, params):
    fp = s.shape[0]
    blk = 1024
    grid = (fp // blk,)
    w5 = jnp.pad(params['Wr5'], ((0, 0), (0, 4)))
    b5 = jnp.pad(params['br5'], ((0, 4),))
    args = [
        jnp.pad(params['Wr0'] * (1.0 / 3.0), ((0, 128), (0, 0))),
        params['br0'].reshape(1, -1),
        params['Wr1'], params['br1'].reshape(1, -1),
        params['Wr2'], params['br2'].reshape(1, -1),
        params['Wr3'], params['br3'].reshape(1, -1),
        params['Wr4'], params['br4'].reshape(1, -1),
        w5, b5.reshape(-1, 1),
    ]

    def wspec(shape):
        return pl.BlockSpec(shape, lambda i: (0, 0))

    return pl.pallas_call(
        _rmlp_body,
        grid=grid,
        in_specs=[pl.BlockSpec((blk, 384), lambda i: (i, 0))] +
                 [wspec(a.shape) for a in args],
        out_specs=pl.BlockSpec((16, blk), lambda i: (0, i)),
        out_shape=jax.ShapeDtypeStruct((16, fp), _F32),
    )(s, *args)


def _sym4_sq_norm(b):
    b00, b01, b02, b03, b11, b12, b13, b22, b23, b33 = b
    c00 = b00 * b00 + b01 * b01 + b02 * b02 + b03 * b03
    c01 = b00 * b01 + b01 * b11 + b02 * b12 + b03 * b13
    c02 = b00 * b02 + b01 * b12 + b02 * b22 + b03 * b23
    c03 = b00 * b03 + b01 * b13 + b02 * b23 + b03 * b33
    c11 = b01 * b01 + b11 * b11 + b12 * b12 + b13 * b13
    c12 = b01 * b02 + b11 * b12 + b12 * b22 + b13 * b23
    c13 = b01 * b03 + b11 * b13 + b12 * b23 + b13 * b33
    c22 = b02 * b02 + b12 * b12 + b22 * b22 + b23 * b23
    c23 = b02 * b03 + b12 * b13 + b22 * b23 + b23 * b33
    c33 = b03 * b03 + b13 * b13 + b23 * b23 + b33 * b33
    tr = c00 + c11 + c22 + c33
    r = 1.0 / jnp.maximum(tr, 1e-30)
    return tuple(v * r for v in (c00, c01, c02, c03, c11, c12, c13, c22, c23, c33))


def _proc_body(f_real, f_ref, p0_ref, p1_ref, p2_ref, r_ref, tp_ref, sc_ref):
    m = [f_ref[k] for k in range(9)]
    t = [f_ref[9 + k] for k in range(3)]
    m00, m01, m02, m10, m11, m12, m20, m21, m22 = m
    fro2 = m00 * m00
    for mi in m[1:]:
        fro2 = fro2 + mi * mi
    sig = jnp.sqrt(3.0 * fro2)
    k00 = m00 + m11 + m22
    k01 = m21 - m12
    k02 = m02 - m20
    k03 = m10 - m01
    k11 = m00 - m11 - m22
    k12 = m01 + m10
    k13 = m02 + m20
    k22 = m11 - m00 - m22
    k23 = m12 + m21
    k33 = m22 - m00 - m11
    r0 = 1.0 / jnp.maximum(4.0 * sig, 1e-30)
    b = tuple(v * r0 for v in (k00 + sig, k01, k02, k03, k11 + sig, k12, k13,
                               k22 + sig, k23, k33 + sig))
    for _ in range(12):
        b = _sym4_sq_norm(b)
    b00, b01, b02, b03, b11, b12, b13, b22, b23, b33 = b
    c01 = b00 >= b11
    a0 = jnp.where(c01, b00, b01)
    a1 = jnp.where(c01, b01, b11)
    a2 = jnp.where(c01, b02, b12)
    a3 = jnp.where(c01, b03, b13)
    da = jnp.where(c01, b00, b11)
    c23 = b22 >= b33
    e0 = jnp.where(c23, b02, b03)
    e1 = jnp.where(c23, b12, b13)
    e2 = jnp.where(c23, b22, b23)
    e3 = jnp.where(c23, b23, b33)
    de = jnp.where(c23, b22, b33)
    cab = da >= de
    qw = jnp.where(cab, a0, e0)
    qx = jnp.where(cab, a1, e1)
    qy = jnp.where(cab, a2, e2)
    qz = jnp.where(cab, a3, e3)
    inn = qw * qw + qx * qx + qy * qy + qz * qz
    s = lax.rsqrt(jnp.maximum(inn, 1e-30))
    w, x, y, z = qw * s, qx * s, qy * s, qz * s
    r00 = 1.0 - 2.0 * (y * y + z * z)
    r01 = 2.0 * (x * y - w * z)
    r02 = 2.0 * (x * z + w * y)
    r10 = 2.0 * (x * y + w * z)
    r11 = 1.0 - 2.0 * (x * x + z * z)
    r12 = 2.0 * (y * z - w * x)
    r20 = 2.0 * (x * z - w * y)
    r21 = 2.0 * (y * z + w * x)
    r22 = 1.0 - 2.0 * (x * x + y * y)
    rm = [r00, r01, r02, r10, r11, r12, r20, r21, r22]
    r_ref[...] = jnp.stack(rm, axis=-1)
    gi = pl.program_id(0)
    row = lax.broadcasted_iota(jnp.int32, (8, 128), 0)
    col = lax.broadcasted_iota(jnp.int32, (8, 128), 1)
    valid = ((gi * 1024 + row * 128 + col) < f_real).astype(_F32)
    ps = [p0_ref, p1_ref, p2_ref]
    tp = [[None] * 3 for _ in range(3)]
    for i in range(3):
        for k in range(3):
            tp[i][k] = (ps[i][:, :, 0] * rm[k] + ps[i][:, :, 1] * rm[3 + k] +
                        ps[i][:, :, 2] * rm[6 + k] + t[k])
    tp_ref[...] = jnp.stack(
        [tp[i][k] for i in range(3) for k in range(3)], axis=-1)
    for k in range(3):
        for i in range(3):
            sc_ref[k * 3 + i] = tp[i][k] * valid
    for i in range(3):
        sc_ref[9 + i] = valid


def _procrustes(ft, p0, p1, p2, f_real):
    nblk = ft.shape[1]
    nb = nblk // 8
    grid = (nb,)
    pspec = pl.BlockSpec((8, 128, 16), lambda i: (i, 0, 0))
    return pl.pallas_call(
        functools.partial(_proc_body, f_real),
        grid=grid,
        in_specs=[pl.BlockSpec((16, 8, 128), lambda i: (0, i, 0)),
                  pspec, pspec, pspec],
        out_specs=[pl.BlockSpec((8, 128, 9), lambda i: (i, 0, 0)),
                   pl.BlockSpec((8, 128, 9), lambda i: (i, 0, 0)),
                   pl.BlockSpec((12, 8, 128), lambda i: (0, i, 0))],
        out_shape=[jax.ShapeDtypeStruct((nblk, 128, 9), _F32),
                   jax.ShapeDtypeStruct((nblk, 128, 9), _F32),
                   jax.ShapeDtypeStruct((12, nblk, 128), _F32)],
    )(ft, p0, p1, p2)


def _comb_body(p_ref, o_ref):
    cnt = jnp.maximum(p_ref[0, 3, :] + p_ref[1, 3, :], 1.0)
    for k in range(3):
        o_ref[k] = (p_ref[0, k, :] + p_ref[1, k, :]) / cnt


def _combine(parts):
    nv = parts.shape[2]
    blk = 2048
    grid = (nv // blk,)
    return pl.pallas_call(
        _comb_body,
        grid=grid,
        in_specs=[pl.BlockSpec((2, 4, blk), lambda i: (0, 0, i))],
        out_specs=pl.BlockSpec((3, blk), lambda i: (0, i)),
        out_shape=jax.ShapeDtypeStruct((3, nv), _F32),
    )(parts)


def _face_gather(x, facep):
    fp = facep.shape[1]
    face_flat = facep.reshape(3 * fp)
    ch = 128
    nw = 32
    fw = fp // nw
    nch = fw // ch
    mesh = plsc.VectorSubcoreMesh(core_axis_name="c", subcore_axis_name="s")

    @functools.partial(
        pl.kernel, mesh=mesh,
        out_type=[jax.ShapeDtypeStruct((fp, 384), _F32),
                  jax.ShapeDtypeStruct((3, fp, 16), _F32)],
        scratch_types=[pltpu.VMEM((ch,), jnp.int32),
                       pltpu.VMEM((ch, 384), _F32),
                       pltpu.VMEM((ch, 384), _F32),
                       pltpu.VMEM((ch, 16), _F32),
                       pltpu.SemaphoreType.DMA],
    )
    def k(x_hbm, face_hbm, s_out, p_out, idx_v, rows_v, acc_v, vrow_v, sem):
        c = lax.axis_index("c")
        s = lax.axis_index("s")
        wid = s * 2 + c

        def chunk(i, carry):
            base = wid * fw + i * ch
            for j in range(3):
                pltpu.sync_copy(face_hbm.at[pl.ds(j * fp + base, ch)], idx_v)
                dst = acc_v if j == 0 else rows_v
                pltpu.async_copy(x_hbm.at[idx_v], dst, sem).wait()

                def row_fix(r, carry2):
                    if j > 0:
                        for g in range(256 // 16):
                            sl = pl.ds(g * 16, 16)
                            acc_v[r, sl] = acc_v[r, sl] + rows_v[r, sl]
                    vrow_v[r, :] = dst[r, pl.ds(256, 16)]
                    return carry2

                lax.fori_loop(0, ch, row_fix, 0)
                pltpu.sync_copy(vrow_v, p_out.at[j, pl.ds(base, ch)])
            pltpu.sync_copy(acc_v, s_out.at[pl.ds(base, ch)])
            return carry

        lax.fori_loop(0, nch, chunk, 0)

    return k(x, face_flat)


def _vertex_scatter(sdata_flat, sidx, nv):
    rp = sdata_flat.shape[0] // 4
    ch = 2048
    half = rp // 2
    nchunk = half // ch
    vt = nv // 16
    mesh = plsc.VectorSubcoreMesh(core_axis_name="c", subcore_axis_name="s")

    @functools.partial(
        pl.kernel, mesh=mesh,
        compiler_params=pltpu.CompilerParams(needs_layout_passes=False),
        out_type=jax.ShapeDtypeStruct((8 * nv,), _F32),
        scratch_types=[pltpu.VMEM((ch,), jnp.int32),
                       pltpu.VMEM((4 * ch,), _F32),
                       pltpu.VMEM((4 * vt,), _F32)],
    )
    def k(data_hbm, idx_hbm, out_hbm, idx_v, dbuf, acc):
        c = lax.axis_index("c")
        s = lax.axis_index("s")
        vbase = s * vt

        def z(i, carry):
            acc[pl.ds(i * 16, 16)] = jnp.zeros((16,), _F32)
            return carry

        lax.fori_loop(0, (4 * vt) // 16, z, 0)

        def chunk(i, carry):
            base = c * half + i * ch
            pltpu.sync_copy(idx_hbm.at[pl.ds(base, ch)], idx_v)
            for kk in range(4):
                pltpu.sync_copy(data_hbm.at[pl.ds(kk * rp + base, ch)],
                                dbuf.at[pl.ds(kk * ch, ch)])

            def g(j, carry2):
                iv = idx_v[pl.ds(j * 16, 16)]
                local = iv - vbase
                msk = (local >= 0) & (local < vt)
                loc = jnp.minimum(jnp.maximum(local, 0), vt - 1)
                for kk in range(4):
                    dk = dbuf[pl.ds(kk * ch + j * 16, 16)]
                    plsc.addupdate_scatter(acc, [loc + kk * vt], dk, mask=msk)
                return carry2

            lax.fori_loop(0, ch // 16, g, 0)
            return carry

        lax.fori_loop(0, nchunk, chunk, 0)
        for kk in range(4):
            pltpu.sync_copy(
                acc.at[pl.ds(kk * vt, vt)],
                out_hbm.at[pl.ds((c * 4 + kk) * nv + s * vt, vt)])

    return k(sdata_flat, sidx)


def kernel(pos, x_in, face, params):
    n = pos.shape[1]
    f = face.shape[1]
    blk = 1024
    npad = ((n + blk - 1) // blk) * blk
    fp = ((f + 4095) // 4096) * 4096

    xp = jnp.pad(x_in.reshape(n, 128), ((0, npad - n), (0, 0)))
    verts = pos.reshape(n, 3)
    vp = jnp.pad(verts, ((0, npad - n), (0, 1)))
    x = _vertex_mlp(xp, vp, params)

    facep = jnp.pad(face, ((0, 0), (0, fp - f)))

    ssum, prism = _face_gather(x, facep)

    fvec_cm = _refine_mlp(ssum, params)

    ft = fvec_cm.reshape(16, fp // 128, 128)
    pr = prism.reshape(3, fp // 128, 128, 16)
    r_rm, tp_rm, out12 = _procrustes(ft, pr[0], pr[1], pr[2], f)

    rmat = r_rm.reshape(fp, 9)[:f].reshape(f, 3, 3)
    tprism = tp_rm.reshape(fp, 9)[:f].reshape(f, 3, 3)

    sdata_flat = out12.reshape(12 * fp)
    sidx = facep.reshape(3 * fp)
    nvp = ((n + 16383) // 16384) * 16384

    parts = _vertex_scatter(sdata_flat, sidx, nvp)
    vcomb = _combine(parts.reshape(2, 4, nvp))

    return vcomb.T[:n].reshape(1, n, 3), tprism, rmat


# speedup vs baseline: 1.2090x; 1.1271x over previous
import functools

import jax
import jax.numpy as jnp
from jax import lax
from jax.experimental import pallas as pl
from jax.experimental.pallas import tpu as pltpu
from jax.experimental.pallas import tpu_sc as plsc

_F32 = jnp.float32
_PREC = lax.Precision.DEFAULT


def _dot(a, b):
    return lax.dot(a, b, precision=_PREC, preferred_element_type=_F32)


def _vmlp_body(x_ref, v_ref, wi, bi, w1, b1, w2, b2, wo, bo, o_ref):
    x = x_ref[...]
    h = jnp.maximum(_dot(x, wi[...]) + bi[...], 0.0)
    h = h + jnp.maximum(_dot(h, w1[...]) + b1[...], 0.0)
    h = h + jnp.maximum(_dot(h, w2[...]) + b2[...], 0.0)
    o_ref[:, 0:256] = _dot(h, wo[...]) + bo[...]
    o_ref[:, 256:384] = jnp.pad(v_ref[...], ((0, 0), (0, 124)))


def _vertex_mlp(xp, vp, params):
    npad = xp.shape[0]
    blk = 1024
    grid = (npad // blk,)

    def wspec(shape):
        return pl.BlockSpec(shape, lambda i: (0, 0))

    args = [
        params['Wd_in'], params['bd_in'].reshape(1, -1),
        params['Wd1'], params['bd1'].reshape(1, -1),
        params['Wd2'], params['bd2'].reshape(1, -1),
        params['Wd_out'], params['bd_out'].reshape(1, -1),
    ]
    return pl.pallas_call(
        _vmlp_body,
        grid=grid,
        in_specs=[pl.BlockSpec((blk, 128), lambda i: (i, 0)),
                  pl.BlockSpec((blk, 4), lambda i: (i, 0))] +
                 [wspec(a.shape) for a in args],
        out_specs=pl.BlockSpec((blk, 384), lambda i: (i, 0)),
        out_shape=jax.ShapeDtypeStruct((npad, 384), _F32),
    )(xp, vp, *args)


def _rmlp_body(s_ref, w0, b0, w1, b1, w2, b2, w3, b3, w4, b4, w5, b5, o_ref):
    h = jnp.maximum(_dot(s_ref[...], w0[...]) + b0[...], 0.0)
    h = jnp.maximum(_dot(h, w1[...]) + b1[...], 0.0)
    h = jnp.maximum(_dot(h, w2[...]) + b2[...], 0.0)
    h = jnp.maximum(_dot(h, w3[...]) + b3[...], 0.0)
    h = jnp.maximum(_dot(h, w4[...]) + b4[...], 0.0)
    ft = lax.dot_general(w5[...], h, (((0,), (1,)), ((), ())),
                         precision=_PREC, preferred_element_type=_F32)
    o_ref[...] = ft + b5[...]


def _refine_mlp(s, params):
    fp = s.shape[0]
    blk = 1024
    grid = (fp // blk,)
    w5 = jnp.pad(params['Wr5'], ((0, 0), (0, 4)))
    b5 = jnp.pad(params['br5'], ((0, 4),))
    args = [
        jnp.pad(params['Wr0'] * (1.0 / 3.0), ((0, 128), (0, 0))),
        params['br0'].reshape(1, -1),
        params['Wr1'], params['br1'].reshape(1, -1),
        params['Wr2'], params['br2'].reshape(1, -1),
        params['Wr3'], params['br3'].reshape(1, -1),
        params['Wr4'], params['br4'].reshape(1, -1),
        w5, b5.reshape(-1, 1),
    ]

    def wspec(shape):
        return pl.BlockSpec(shape, lambda i: (0, 0))

    return pl.pallas_call(
        _rmlp_body,
        grid=grid,
        in_specs=[pl.BlockSpec((blk, 384), lambda i: (i, 0))] +
                 [wspec(a.shape) for a in args],
        out_specs=pl.BlockSpec((16, blk), lambda i: (0, i)),
        out_shape=jax.ShapeDtypeStruct((16, fp), _F32),
    )(s, *args)


def _sym4_sq_norm(b):
    b00, b01, b02, b03, b11, b12, b13, b22, b23, b33 = b
    c00 = b00 * b00 + b01 * b01 + b02 * b02 + b03 * b03
    c01 = b00 * b01 + b01 * b11 + b02 * b12 + b03 * b13
    c02 = b00 * b02 + b01 * b12 + b02 * b22 + b03 * b23
    c03 = b00 * b03 + b01 * b13 + b02 * b23 + b03 * b33
    c11 = b01 * b01 + b11 * b11 + b12 * b12 + b13 * b13
    c12 = b01 * b02 + b11 * b12 + b12 * b22 + b13 * b23
    c13 = b01 * b03 + b11 * b13 + b12 * b23 + b13 * b33
    c22 = b02 * b02 + b12 * b12 + b22 * b22 + b23 * b23
    c23 = b02 * b03 + b12 * b13 + b22 * b23 + b23 * b33
    c33 = b03 * b03 + b13 * b13 + b23 * b23 + b33 * b33
    tr = c00 + c11 + c22 + c33
    r = 1.0 / jnp.maximum(tr, 1e-30)
    return tuple(v * r for v in (c00, c01, c02, c03, c11, c12, c13, c22, c23, c33))


def _proc_body(f_real, f_ref, p_ref, r_ref, tp_ref, sc_ref):
    m = [f_ref[k] for k in range(9)]
    t = [f_ref[9 + k] for k in range(3)]
    m00, m01, m02, m10, m11, m12, m20, m21, m22 = m
    fro2 = m00 * m00
    for mi in m[1:]:
        fro2 = fro2 + mi * mi
    sig = jnp.sqrt(3.0 * fro2)
    k00 = m00 + m11 + m22
    k01 = m21 - m12
    k02 = m02 - m20
    k03 = m10 - m01
    k11 = m00 - m11 - m22
    k12 = m01 + m10
    k13 = m02 + m20
    k22 = m11 - m00 - m22
    k23 = m12 + m21
    k33 = m22 - m00 - m11
    r0 = 1.0 / jnp.maximum(4.0 * sig, 1e-30)
    b = tuple(v * r0 for v in (k00 + sig, k01, k02, k03, k11 + sig, k12, k13,
                               k22 + sig, k23, k33 + sig))
    for _ in range(12):
        b = _sym4_sq_norm(b)
    b00, b01, b02, b03, b11, b12, b13, b22, b23, b33 = b
    c01 = b00 >= b11
    a0 = jnp.where(c01, b00, b01)
    a1 = jnp.where(c01, b01, b11)
    a2 = jnp.where(c01, b02, b12)
    a3 = jnp.where(c01, b03, b13)
    da = jnp.where(c01, b00, b11)
    c23 = b22 >= b33
    e0 = jnp.where(c23, b02, b03)
    e1 = jnp.where(c23, b12, b13)
    e2 = jnp.where(c23, b22, b23)
    e3 = jnp.where(c23, b23, b33)
    de = jnp.where(c23, b22, b33)
    cab = da >= de
    qw = jnp.where(cab, a0, e0)
    qx = jnp.where(cab, a1, e1)
    qy = jnp.where(cab, a2, e2)
    qz = jnp.where(cab, a3, e3)
    inn = qw * qw + qx * qx + qy * qy + qz * qz
    s = lax.rsqrt(jnp.maximum(inn, 1e-30))
    w, x, y, z = qw * s, qx * s, qy * s, qz * s
    r00 = 1.0 - 2.0 * (y * y + z * z)
    r01 = 2.0 * (x * y - w * z)
    r02 = 2.0 * (x * z + w * y)
    r10 = 2.0 * (x * y + w * z)
    r11 = 1.0 - 2.0 * (x * x + z * z)
    r12 = 2.0 * (y * z - w * x)
    r20 = 2.0 * (x * z - w * y)
    r21 = 2.0 * (y * z + w * x)
    r22 = 1.0 - 2.0 * (x * x + y * y)
    rm = [r00, r01, r02, r10, r11, r12, r20, r21, r22]
    for k in range(9):
        r_ref[k] = rm[k]
    for k in range(9, 16):
        r_ref[k] = jnp.zeros((8, 128), _F32)
    gi = pl.program_id(0)
    row = lax.broadcasted_iota(jnp.int32, (8, 128), 0)
    col = lax.broadcasted_iota(jnp.int32, (8, 128), 1)
    valid = ((gi * 1024 + row * 128 + col) < f_real).astype(_F32)
    tp = [[None] * 3 for _ in range(3)]
    for i in range(3):
        pik = [p_ref[i, k] for k in range(3)]
        for k in range(3):
            tp[i][k] = (pik[0] * rm[k] + pik[1] * rm[3 + k] +
                        pik[2] * rm[6 + k] + t[k])
    for i in range(3):
        for k in range(3):
            tp_ref[3 * i + k] = tp[i][k]
    for k in range(9, 16):
        tp_ref[k] = jnp.zeros((8, 128), _F32)
    for k in range(3):
        for i in range(3):
            sc_ref[k * 3 + i] = tp[i][k] * valid
    for i in range(3):
        sc_ref[9 + i] = valid


def _procrustes(ft, pcm, f_real):
    nblk = ft.shape[1]
    nb = nblk // 8
    grid = (nb,)
    return pl.pallas_call(
        functools.partial(_proc_body, f_real),
        grid=grid,
        in_specs=[pl.BlockSpec((16, 8, 128), lambda i: (0, i, 0)),
                  pl.BlockSpec((3, 16, 8, 128), lambda i: (0, 0, i, 0))],
        out_specs=[pl.BlockSpec((16, 8, 128), lambda i: (0, i, 0)),
                   pl.BlockSpec((16, 8, 128), lambda i: (0, i, 0)),
                   pl.BlockSpec((12, 8, 128), lambda i: (0, i, 0))],
        out_shape=[jax.ShapeDtypeStruct((16, nblk, 128), _F32),
                   jax.ShapeDtypeStruct((16, nblk, 128), _F32),
                   jax.ShapeDtypeStruct((12, nblk, 128), _F32)],
    )(ft, pcm)


def _tx_cm_body(p_ref, eye_ref, o_ref):
    for kk in range(4):
        blk = p_ref[0, pl.ds(kk * 128, 128), :]
        o_ref[0, :, pl.ds(kk * 128, 128)] = lax.dot_general(
            blk, eye_ref[...], (((0,), (0,)), ((), ())),
            precision=lax.Precision.HIGHEST, preferred_element_type=_F32)


def _prism_to_cm(prism):
    fp = prism.shape[1]
    eye = jnp.eye(128, dtype=_F32)
    return pl.pallas_call(
        _tx_cm_body,
        grid=(3, fp // 512),
        in_specs=[pl.BlockSpec((1, 512, 16), lambda j, i: (j, i, 0)),
                  pl.BlockSpec((128, 128), lambda j, i: (0, 0))],
        out_specs=pl.BlockSpec((1, 16, 512), lambda j, i: (j, 0, i)),
        out_shape=jax.ShapeDtypeStruct((3, 16, fp), _F32),
    )(prism, eye)


def _tx_rm_body(a_ref, b_ref, eye_ref, oa_ref, ob_ref):
    for kk in range(4):
        sl = pl.ds(kk * 128, 128)
        oa_ref[sl, :] = lax.dot_general(
            a_ref[:, sl], eye_ref[...], (((0,), (0,)), ((), ())),
            precision=lax.Precision.HIGHEST, preferred_element_type=_F32)
        ob_ref[sl, :] = lax.dot_general(
            b_ref[:, sl], eye_ref[...], (((0,), (0,)), ((), ())),
            precision=lax.Precision.HIGHEST, preferred_element_type=_F32)


def _cm_to_rm(a, b):
    fp = a.shape[1]
    eye = jnp.eye(16, dtype=_F32)
    spec = pl.BlockSpec((16, 512), lambda i: (0, i))
    ospec = pl.BlockSpec((512, 16), lambda i: (i, 0))
    return pl.pallas_call(
        _tx_rm_body,
        grid=(fp // 512,),
        in_specs=[spec, spec, pl.BlockSpec((16, 16), lambda i: (0, 0))],
        out_specs=[ospec, ospec],
        out_shape=[jax.ShapeDtypeStruct((fp, 16), _F32),
                   jax.ShapeDtypeStruct((fp, 16), _F32)],
    )(a, b, eye)


def _comb_body(p_ref, o_ref):
    cnt = jnp.maximum(p_ref[0, 3, :] + p_ref[1, 3, :], 1.0)
    for k in range(3):
        o_ref[k] = (p_ref[0, k, :] + p_ref[1, k, :]) / cnt


def _combine(parts):
    nv = parts.shape[2]
    blk = 2048
    grid = (nv // blk,)
    return pl.pallas_call(
        _comb_body,
        grid=grid,
        in_specs=[pl.BlockSpec((2, 4, blk), lambda i: (0, 0, i))],
        out_specs=pl.BlockSpec((3, blk), lambda i: (0, i)),
        out_shape=jax.ShapeDtypeStruct((3, nv), _F32),
    )(parts)


def _face_gather(x, facep):
    fp = facep.shape[1]
    face_flat = facep.reshape(3 * fp)
    ch = 128
    nw = 32
    fw = fp // nw
    nch = fw // ch
    mesh = plsc.VectorSubcoreMesh(core_axis_name="c", subcore_axis_name="s")

    @functools.partial(
        pl.kernel, mesh=mesh,
        out_type=[jax.ShapeDtypeStruct((fp, 384), _F32),
                  jax.ShapeDtypeStruct((3, fp, 16), _F32)],
        scratch_types=[pltpu.VMEM((ch,), jnp.int32),
                       pltpu.VMEM((ch, 384), _F32),
                       pltpu.VMEM((ch, 384), _F32),
                       pltpu.VMEM((ch, 16), _F32),
                       pltpu.SemaphoreType.DMA],
    )
    def k(x_hbm, face_hbm, s_out, p_out, idx_v, rows_v, acc_v, vrow_v, sem):
        c = lax.axis_index("c")
        s = lax.axis_index("s")
        wid = s * 2 + c

        def chunk(i, carry):
            base = wid * fw + i * ch
            for j in range(3):
                pltpu.sync_copy(face_hbm.at[pl.ds(j * fp + base, ch)], idx_v)
                dst = acc_v if j == 0 else rows_v
                pltpu.async_copy(x_hbm.at[idx_v], dst, sem).wait()

                def row_fix(r, carry2):
                    if j > 0:
                        for g in range(256 // 16):
                            sl = pl.ds(g * 16, 16)
                            acc_v[r, sl] = acc_v[r, sl] + rows_v[r, sl]
                    vrow_v[r, :] = dst[r, pl.ds(256, 16)]
                    return carry2

                lax.fori_loop(0, ch, row_fix, 0)
                pltpu.sync_copy(vrow_v, p_out.at[j, pl.ds(base, ch)])
            pltpu.sync_copy(acc_v, s_out.at[pl.ds(base, ch)])
            return carry

        lax.fori_loop(0, nch, chunk, 0)

    return k(x, face_flat)


def _vertex_scatter(sdata_flat, sidx, nv):
    rp = sdata_flat.shape[0] // 4
    ch = 2048
    half = rp // 2
    nchunk = half // ch
    vt = nv // 16
    mesh = plsc.VectorSubcoreMesh(core_axis_name="c", subcore_axis_name="s")

    @functools.partial(
        pl.kernel, mesh=mesh,
        compiler_params=pltpu.CompilerParams(needs_layout_passes=False),
        out_type=jax.ShapeDtypeStruct((8 * nv,), _F32),
        scratch_types=[pltpu.VMEM((ch,), jnp.int32),
                       pltpu.VMEM((4 * ch,), _F32),
                       pltpu.VMEM((4 * vt,), _F32)],
    )
    def k(data_hbm, idx_hbm, out_hbm, idx_v, dbuf, acc):
        c = lax.axis_index("c")
        s = lax.axis_index("s")
        vbase = s * vt

        def z(i, carry):
            acc[pl.ds(i * 16, 16)] = jnp.zeros((16,), _F32)
            return carry

        lax.fori_loop(0, (4 * vt) // 16, z, 0)

        def chunk(i, carry):
            base = c * half + i * ch
            pltpu.sync_copy(idx_hbm.at[pl.ds(base, ch)], idx_v)
            for kk in range(4):
                pltpu.sync_copy(data_hbm.at[pl.ds(kk * rp + base, ch)],
                                dbuf.at[pl.ds(kk * ch, ch)])

            def g(j, carry2):
                iv = idx_v[pl.ds(j * 16, 16)]
                local = iv - vbase
                msk = (local >= 0) & (local < vt)
                loc = jnp.minimum(jnp.maximum(local, 0), vt - 1)
                for kk in range(4):
                    dk = dbuf[pl.ds(kk * ch + j * 16, 16)]
                    plsc.addupdate_scatter(acc, [loc + kk * vt], dk, mask=msk)
                return carry2

            lax.fori_loop(0, ch // 16, g, 0)
            return carry

        lax.fori_loop(0, nchunk, chunk, 0)
        for kk in range(4):
            pltpu.sync_copy(
                acc.at[pl.ds(kk * vt, vt)],
                out_hbm.at[pl.ds((c * 4 + kk) * nv + s * vt, vt)])

    return k(sdata_flat, sidx)


def kernel(pos, x_in, face, params):
    n = pos.shape[1]
    f = face.shape[1]
    blk = 1024
    npad = ((n + blk - 1) // blk) * blk
    fp = ((f + 4095) // 4096) * 4096

    xp = jnp.pad(x_in.reshape(n, 128), ((0, npad - n), (0, 0)))
    verts = pos.reshape(n, 3)
    vp = jnp.pad(verts, ((0, npad - n), (0, 1)))
    x = _vertex_mlp(xp, vp, params)

    facep = jnp.pad(face, ((0, 0), (0, fp - f)))

    ssum, prism = _face_gather(x, facep)

    fvec_cm = _refine_mlp(ssum, params)

    ft = fvec_cm.reshape(16, fp // 128, 128)
    pcm = _prism_to_cm(prism).reshape(3, 16, fp // 128, 128)
    r16, tp16, out12 = _procrustes(ft, pcm, f)
    r_rm, tp_rm = _cm_to_rm(r16.reshape(16, fp), tp16.reshape(16, fp))

    rmat = r_rm[:f, :9].reshape(f, 3, 3)
    tprism = tp_rm[:f, :9].reshape(f, 3, 3)

    sdata_flat = out12.reshape(12 * fp)
    sidx = facep.reshape(3 * fp)
    nvp = ((n + 16383) // 16384) * 16384

    parts = _vertex_scatter(sdata_flat, sidx, nvp)
    vcomb = _combine(parts.reshape(2, 4, nvp))

    return vcomb.T[:n].reshape(1, n, 3), tprism, rmat
